# baseline (speedup 1.0000x reference)
import functools

import jax
import jax.numpy as jnp
import numpy as np
from jax import lax
from jax.experimental import pallas as pl
from jax.experimental.pallas import tpu as pltpu
from jax.experimental.pallas import tpu_sc as plsc

D_MODEL = 768
N_HEADS = 12
HEAD_DIM = 64
D_FF = 1536
N_EXP = 8
SEQ = 2048
SBLK = 256
NBLK = SEQ // SBLK
EPS = 1e-6
_SQRT2 = 1.4142135623730951

BLK = 256
N_ITEMS = SEQ * 2
P_MAX = 5888
NB_FFN = P_MAX // BLK
N_TILES = 32
IPT = N_ITEMS // N_TILES
TPT = SEQ // N_TILES


def _gelu(v):
    return 0.5 * v * (1.0 + jax.lax.erf(v / _SQRT2))


def _rms(v):
    return v * jax.lax.rsqrt(jnp.mean(v * v, axis=-1, keepdims=True) + EPS)


def _attn_body(x_ref, qW_ref, qWs_ref, kvW_ref, ct_ref, st_ref, ao_ref):
    xn = _rms(x_ref[...]).astype(jnp.bfloat16)
    dn = (((1,), (1,)), ((), ()))
    q = jax.lax.dot_general(xn, qW_ref[...], dn, preferred_element_type=jnp.float32)
    qs = jax.lax.dot_general(xn, qWs_ref[...], dn, preferred_element_type=jnp.float32)
    rq = q * ct_ref[...] + qs * st_ref[...]
    kv = jax.lax.dot_general(xn, kvW_ref[...], dn, preferred_element_type=jnp.float32)
    k_cat = jnp.concatenate(
        [kv[:, j * 128: j * 128 + 64] for j in range(N_HEADS)], axis=1)
    v_cat = jnp.concatenate(
        [kv[:, j * 128 + 64: j * 128 + 128] for j in range(N_HEADS)], axis=1)
    r768 = jax.lax.broadcasted_iota(jnp.int32, (D_MODEL, N_HEADS), 0)
    c12 = jax.lax.broadcasted_iota(jnp.int32, (D_MODEL, N_HEADS), 1)
    G = (r768 // HEAD_DIM == c12).astype(jnp.float32)
    r12 = jax.lax.broadcasted_iota(jnp.int32, (N_HEADS, D_MODEL), 0)
    c768 = jax.lax.broadcasted_iota(jnp.int32, (N_HEADS, D_MODEL), 1)
    E = (c768 // HEAD_DIM == r12).astype(jnp.float32)
    c64 = jax.lax.broadcasted_iota(jnp.int32, (D_MODEL, HEAD_DIM), 1)
    r768b = jax.lax.broadcasted_iota(jnp.int32, (D_MODEL, HEAD_DIM), 0)
    Hm = (r768b % HEAD_DIM == c64).astype(jnp.float32)
    q3 = rq.reshape(SBLK, N_HEADS, HEAD_DIM)
    k3 = k_cat.reshape(SBLK, N_HEADS, HEAD_DIM)
    v3 = v_cat.reshape(SBLK, N_HEADS, HEAD_DIM)
    att = jax.lax.dot_general(q3, k3, (((2,), (2,)), ((0,), (0,))),
                              preferred_element_type=jnp.float32) * 0.125
    att = att - jnp.max(att, axis=2, keepdims=True)
    p = jnp.exp(att)
    att = p / jnp.sum(p, axis=2, keepdims=True)
    ao = jax.lax.dot_general(att, v3, (((2,), (1,)), ((0,), (0,))),
                             preferred_element_type=jnp.float32)
    for i in range(N_HEADS):
        ao_ref[i] = ao[:, i, :]


def _oroute_body(ao_ref, x_ref, oW_ref, gW_ref, h_ref, hn_ref, route_ref):
    dn = (((1,), (1,)), ((), ()))
    h = jax.lax.dot_general(ao_ref[...].astype(jnp.bfloat16), oW_ref[...], dn,
                            preferred_element_type=jnp.float32) + x_ref[...]
    h_ref[...] = h
    hn = _rms(h)
    hn_ref[...] = hn
    gl = jax.lax.dot_general(hn, gW_ref[...], dn, preferred_element_type=jnp.float32)
    gl = gl - jnp.max(gl, axis=1, keepdims=True)
    pe = jnp.exp(gl)
    gw = pe / jnp.sum(pe, axis=1, keepdims=True)
    io = jax.lax.broadcasted_iota(jnp.int32, gw.shape, 1)
    m1 = jnp.max(gw, axis=1, keepdims=True)
    i1 = jnp.min(jnp.where(gw == m1, io, N_EXP), axis=1, keepdims=True)
    gm = jnp.where(io == i1, -jnp.inf, gw)
    m2 = jnp.max(gm, axis=1, keepdims=True)
    i2 = jnp.min(jnp.where((gw == m2) & (io != i1), io, N_EXP),
                 axis=1, keepdims=True)
    s12 = m1 + m2
    w1n = m1 / s12
    w2n = m2 / s12
    route_ref[...] = jnp.where(
        io == 0, i1.astype(jnp.float32),
        jnp.where(io == 1, i2.astype(jnp.float32),
                  jnp.where(io == 2, w1n, jnp.where(io == 3, w2n, 0.0))))


@functools.cache
def _make_sc_dispatch():
    mesh = plsc.VectorSubcoreMesh(core_axis_name="c", subcore_axis_name="s")

    @functools.partial(
        pl.kernel, mesh=mesh,
        out_type=jax.ShapeDtypeStruct((P_MAX, D_MODEL), jnp.float32),
        scratch_types=[
            pltpu.VMEM((IPT,), jnp.int32),
            pltpu.VMEM((IPT,), jnp.int32),
            pltpu.VMEM((IPT, D_MODEL), jnp.float32),
            pltpu.SemaphoreType.DMA,
            pltpu.SemaphoreType.DMA,
        ],
    )
    def _sc_dispatch(hn_hbm, pos_hbm, disp_hbm, idx_v, pos_v, rows_v, sem1, sem2):
        wid = lax.axis_index("s") * 2 + lax.axis_index("c")
        tok_base = wid * TPT
        lane = jax.lax.iota(jnp.int32, 16)
        half = jax.lax.shift_right_logical(lane, 1)
        for c in range(IPT // 16):
            idx_v[pl.ds(c * 16, 16)] = tok_base + c * 8 + half
        pltpu.sync_copy(pos_hbm.at[pl.ds(wid * IPT, IPT)], pos_v)
        pltpu.async_copy(hn_hbm.at[idx_v], rows_v, sem1).wait()
        pltpu.async_copy(rows_v, disp_hbm.at[pos_v], sem2).wait()

    return _sc_dispatch


@functools.cache
def _make_sc_combine():
    mesh = plsc.VectorSubcoreMesh(core_axis_name="c", subcore_axis_name="s")

    @functools.partial(
        pl.kernel, mesh=mesh,
        out_type=[jax.ShapeDtypeStruct((SEQ, D_MODEL), jnp.float32),
                  jax.ShapeDtypeStruct((SEQ, D_MODEL), jnp.float32)],
        scratch_types=[
            pltpu.VMEM((TPT,), jnp.int32),
            pltpu.VMEM((TPT, D_MODEL), jnp.float32),
            pltpu.SemaphoreType.DMA,
        ],
    )
    def _sc_combine(y_hbm, p0_hbm, p1_hbm, y0_hbm, y1_hbm, idx_v, rows_v, sem):
        wid = lax.axis_index("s") * 2 + lax.axis_index("c")
        base = wid * TPT
        pltpu.sync_copy(p0_hbm.at[pl.ds(base, TPT)], idx_v)
        pltpu.async_copy(y_hbm.at[idx_v], rows_v, sem).wait()
        pltpu.sync_copy(rows_v, y0_hbm.at[pl.ds(base, TPT)])
        pltpu.sync_copy(p1_hbm.at[pl.ds(base, TPT)], idx_v)
        pltpu.async_copy(y_hbm.at[idx_v], rows_v, sem).wait()
        pltpu.sync_copy(rows_v, y1_hbm.at[pl.ds(base, TPT)])

    return _sc_combine


def _ffn_body(be_ref, disp_ref, w1_ref, w3_ref, w2_ref, y_ref):
    b = pl.program_id(0)
    be = be_ref[b]

    @pl.when(be < N_EXP)
    def _go():
        xb = disp_ref[...].astype(jnp.bfloat16)
        dn = (((1,), (1,)), ((), ()))
        g1 = jax.lax.dot_general(xb, w1_ref[0], dn,
                                 preferred_element_type=jnp.float32)
        g3 = jax.lax.dot_general(xb, w3_ref[0], dn,
                                 preferred_element_type=jnp.float32)
        hcur = (_gelu(g1) * g3).astype(jnp.bfloat16)
        y_ref[...] = jax.lax.dot_general(hcur, w2_ref[0], dn,
                                         preferred_element_type=jnp.float32)


def _final_body(hn_ref, h_ref, route_ref, y0_ref, y1_ref,
                sw1_ref, sw2_ref, sw3_ref, out_ref):
    hn = hn_ref[...].astype(jnp.bfloat16)
    dn = (((1,), (1,)), ((), ()))
    s1 = jax.lax.dot_general(hn, sw1_ref[...], dn, preferred_element_type=jnp.float32)
    s3 = jax.lax.dot_general(hn, sw3_ref[...], dn, preferred_element_type=jnp.float32)
    sh = (_gelu(s1) * s3).astype(jnp.bfloat16)
    shared = jax.lax.dot_general(sh, sw2_ref[...], dn,
                                 preferred_element_type=jnp.float32)
    route = route_ref[...]
    out_ref[...] = (h_ref[...] + shared
                    + route[:, 2:3] * y0_ref[...]
                    + route[:, 3:4] * y1_ref[...])


def _rope_tabs():
    theta = 1.0 / (10000.0 ** (np.arange(0, HEAD_DIM, 2).astype(np.float32)
                               / HEAD_DIM))
    pos = np.arange(SEQ).astype(np.float32)
    fr = np.outer(pos, theta)
    cos, sin = np.cos(fr), np.sin(fr)
    ct = np.zeros((SEQ, D_MODEL), np.float32)
    st = np.zeros((SEQ, D_MODEL), np.float32)
    for h in range(N_HEADS):
        ct[:, h * 64 + 0:h * 64 + 64:2] = cos
        ct[:, h * 64 + 1:h * 64 + 64:2] = cos
        st[:, h * 64 + 0:h * 64 + 64:2] = -sin
        st[:, h * 64 + 1:h * 64 + 64:2] = sin
    return jnp.asarray(ct), jnp.asarray(st)


_CT, _ST = None, None


def kernel(x, q_W, kv_W, o_W, gate_W, w1, w2, w3, sw1, sw2, sw3):
    global _CT, _ST
    if _CT is None:
        _CT, _ST = _rope_tabs()
    B, S, D = x.shape
    x2 = x.reshape(S, D)
    perm = np.arange(D_MODEL).reshape(-1, 2)[:, ::-1].reshape(-1)
    q_Ws = q_W[perm]

    ao3 = pl.pallas_call(
        _attn_body,
        grid=(NBLK,),
        in_specs=[
            pl.BlockSpec((SBLK, D), lambda b: (b, 0)),
            pl.BlockSpec((D, D), lambda b: (0, 0)),
            pl.BlockSpec((D, D), lambda b: (0, 0)),
            pl.BlockSpec((2 * D, D), lambda b: (0, 0)),
            pl.BlockSpec((SBLK, D), lambda b: (b, 0)),
            pl.BlockSpec((SBLK, D), lambda b: (b, 0)),
        ],
        out_specs=pl.BlockSpec((N_HEADS, SBLK, HEAD_DIM), lambda b: (0, b, 0)),
        out_shape=jax.ShapeDtypeStruct((N_HEADS, SEQ, HEAD_DIM), jnp.float32),
    )(x2, q_W.astype(jnp.bfloat16), q_Ws.astype(jnp.bfloat16),
      kv_W.astype(jnp.bfloat16), _CT, _ST)
    ao_scr = ao3.reshape(S, D)

    h, hn, route = pl.pallas_call(
        _oroute_body,
        grid=(NBLK,),
        in_specs=[
            pl.BlockSpec((SBLK, D), lambda b: (b, 0)),
            pl.BlockSpec((SBLK, D), lambda b: (b, 0)),
            pl.BlockSpec((D, D), lambda b: (0, 0)),
            pl.BlockSpec((N_EXP, D), lambda b: (0, 0)),
        ],
        out_specs=[
            pl.BlockSpec((SBLK, D), lambda b: (b, 0)),
            pl.BlockSpec((SBLK, D), lambda b: (b, 0)),
            pl.BlockSpec((SBLK, N_EXP), lambda b: (b, 0)),
        ],
        out_shape=[
            jax.ShapeDtypeStruct((SEQ, D), jnp.float32),
            jax.ShapeDtypeStruct((SEQ, D), jnp.float32),
            jax.ShapeDtypeStruct((SEQ, N_EXP), jnp.float32),
        ],
    )(ao_scr, x2, o_W.astype(jnp.bfloat16), gate_W)

    i1 = route[:, 0].astype(jnp.int32)
    i2 = route[:, 1].astype(jnp.int32)
    e_flat = jnp.stack([i1, i2], axis=1).reshape(-1)
    oh = (e_flat[:, None] == jnp.arange(N_EXP, dtype=jnp.int32)).astype(jnp.int32)
    cum = jnp.cumsum(oh, axis=0)
    counts = cum[-1]
    padded = ((counts + BLK - 1) // BLK) * BLK
    base = jnp.concatenate([jnp.zeros((1,), jnp.int32),
                            jnp.cumsum(padded)[:-1].astype(jnp.int32)])
    rank = jnp.sum(cum * oh, axis=1) - 1
    base_sel = jnp.sum(base[None, :] * oh, axis=1)
    pos = (base_sel + rank).astype(jnp.int32)
    pos2 = pos.reshape(SEQ, 2)
    p0 = pos2[:, 0]
    p1 = pos2[:, 1]
    block_expert = (jnp.sum(
        (jnp.arange(NB_FFN, dtype=jnp.int32)[:, None] * BLK) >= base[None, :],
        axis=1) - 1).astype(jnp.int32)
    nb_used = jnp.sum(padded) // BLK
    block_expert = jnp.where(jnp.arange(NB_FFN) < nb_used, block_expert, N_EXP)

    disp = _make_sc_dispatch()(hn, pos)

    grid_spec = pltpu.PrefetchScalarGridSpec(
        num_scalar_prefetch=1,
        grid=(NB_FFN,),
        in_specs=[
            pl.BlockSpec((BLK, D), lambda b, be: (b, 0)),
            pl.BlockSpec((1, D_FF, D), lambda b, be: (jnp.minimum(be[b], N_EXP - 1), 0, 0)),
            pl.BlockSpec((1, D_FF, D), lambda b, be: (jnp.minimum(be[b], N_EXP - 1), 0, 0)),
            pl.BlockSpec((1, D, D_FF), lambda b, be: (jnp.minimum(be[b], N_EXP - 1), 0, 0)),
        ],
        out_specs=pl.BlockSpec((BLK, D), lambda b, be: (b, 0)),
    )
    y = pl.pallas_call(
        _ffn_body,
        grid_spec=grid_spec,
        out_shape=jax.ShapeDtypeStruct((P_MAX, D), jnp.float32),
    )(block_expert, disp, w1.astype(jnp.bfloat16), w3.astype(jnp.bfloat16),
      w2.astype(jnp.bfloat16))

    y0s, y1s = _make_sc_combine()(y, p0, p1)

    out = pl.pallas_call(
        _final_body,
        grid=(NBLK,),
        in_specs=[
            pl.BlockSpec((SBLK, D), lambda b: (b, 0)),
            pl.BlockSpec((SBLK, D), lambda b: (b, 0)),
            pl.BlockSpec((SBLK, N_EXP), lambda b: (b, 0)),
            pl.BlockSpec((SBLK, D), lambda b: (b, 0)),
            pl.BlockSpec((SBLK, D), lambda b: (b, 0)),
            pl.BlockSpec((D_FF, D), lambda b: (0, 0)),
            pl.BlockSpec((D, D_FF), lambda b: (0, 0)),
            pl.BlockSpec((D_FF, D), lambda b: (0, 0)),
        ],
        out_specs=pl.BlockSpec((SBLK, D), lambda b: (b, 0)),
        out_shape=jax.ShapeDtypeStruct((SEQ, D), jnp.float32),
    )(hn, h, route, y0s, y1s,
      sw1.astype(jnp.bfloat16), sw2.astype(jnp.bfloat16),
      sw3.astype(jnp.bfloat16))

    return out.reshape(B, S, D)


# speedup vs baseline: 1.1801x; 1.1801x over previous
import functools

import jax
import jax.numpy as jnp
import numpy as np
from jax import lax
from jax.experimental import pallas as pl
from jax.experimental.pallas import tpu as pltpu
from jax.experimental.pallas import tpu_sc as plsc

D_MODEL = 768
N_HEADS = 12
HEAD_DIM = 64
D_FF = 1536
N_EXP = 8
SEQ = 2048
SBLK = 256
NBLK = SEQ // SBLK
EPS = 1e-6
_SQRT2 = 1.4142135623730951

BLK = 256
N_ITEMS = SEQ * 2
P_MAX = 5888
NB_FFN = P_MAX // BLK
N_TILES = 32
IPT = N_ITEMS // N_TILES
TPT = SEQ // N_TILES


def _gelu(v):
    return 0.5 * v * (1.0 + jax.lax.erf(v / _SQRT2))


def _rms(v):
    return v * jax.lax.rsqrt(jnp.mean(v * v, axis=-1, keepdims=True) + EPS)


def _attn_body(x_ref, qW_ref, qWs_ref, kvW_ref, ct_ref, st_ref, ao_ref):
    xn = _rms(x_ref[...]).astype(jnp.bfloat16)
    dn = (((1,), (1,)), ((), ()))
    q = jax.lax.dot_general(xn, qW_ref[...], dn, preferred_element_type=jnp.float32)
    qs = jax.lax.dot_general(xn, qWs_ref[...], dn, preferred_element_type=jnp.float32)
    rq = q * ct_ref[...] + qs * st_ref[...]
    kv = jax.lax.dot_general(xn, kvW_ref[...], dn, preferred_element_type=jnp.float32)
    k_cat = jnp.concatenate(
        [kv[:, j * 128: j * 128 + 64] for j in range(N_HEADS)], axis=1)
    v_cat = jnp.concatenate(
        [kv[:, j * 128 + 64: j * 128 + 128] for j in range(N_HEADS)], axis=1)
    r768 = jax.lax.broadcasted_iota(jnp.int32, (D_MODEL, N_HEADS), 0)
    c12 = jax.lax.broadcasted_iota(jnp.int32, (D_MODEL, N_HEADS), 1)
    G = (r768 // HEAD_DIM == c12).astype(jnp.float32)
    r12 = jax.lax.broadcasted_iota(jnp.int32, (N_HEADS, D_MODEL), 0)
    c768 = jax.lax.broadcasted_iota(jnp.int32, (N_HEADS, D_MODEL), 1)
    E = (c768 // HEAD_DIM == r12).astype(jnp.float32)
    c64 = jax.lax.broadcasted_iota(jnp.int32, (D_MODEL, HEAD_DIM), 1)
    r768b = jax.lax.broadcasted_iota(jnp.int32, (D_MODEL, HEAD_DIM), 0)
    Hm = (r768b % HEAD_DIM == c64).astype(jnp.float32)
    q3 = rq.reshape(SBLK, N_HEADS, HEAD_DIM)
    k3 = k_cat.reshape(SBLK, N_HEADS, HEAD_DIM)
    v3 = v_cat.reshape(SBLK, N_HEADS, HEAD_DIM)
    att = jax.lax.dot_general(q3, k3, (((2,), (2,)), ((0,), (0,))),
                              preferred_element_type=jnp.float32) * 0.125
    att = att - jnp.max(att, axis=2, keepdims=True)
    p = jnp.exp(att)
    att = p / jnp.sum(p, axis=2, keepdims=True)
    ao = jax.lax.dot_general(att, v3, (((2,), (1,)), ((0,), (0,))),
                             preferred_element_type=jnp.float32)
    for i in range(N_HEADS):
        ao_ref[i] = ao[:, i, :]


def _oroute_body(ao_ref, x_ref, oW_ref, gW_ref, h_ref, hn_ref, route_ref):
    dn = (((1,), (1,)), ((), ()))
    h = jax.lax.dot_general(ao_ref[...].astype(jnp.bfloat16), oW_ref[...], dn,
                            preferred_element_type=jnp.float32) + x_ref[...]
    h_ref[...] = h
    hn = _rms(h)
    hn_ref[...] = hn
    gl = jax.lax.dot_general(hn, gW_ref[...], dn, preferred_element_type=jnp.float32)
    gl = gl - jnp.max(gl, axis=1, keepdims=True)
    pe = jnp.exp(gl)
    gw = pe / jnp.sum(pe, axis=1, keepdims=True)
    io = jax.lax.broadcasted_iota(jnp.int32, gw.shape, 1)
    m1 = jnp.max(gw, axis=1, keepdims=True)
    i1 = jnp.min(jnp.where(gw == m1, io, N_EXP), axis=1, keepdims=True)
    gm = jnp.where(io == i1, -jnp.inf, gw)
    m2 = jnp.max(gm, axis=1, keepdims=True)
    i2 = jnp.min(jnp.where((gw == m2) & (io != i1), io, N_EXP),
                 axis=1, keepdims=True)
    s12 = m1 + m2
    w1n = m1 / s12
    w2n = m2 / s12
    route_ref[...] = jnp.where(
        io == 0, i1.astype(jnp.float32),
        jnp.where(io == 1, i2.astype(jnp.float32),
                  jnp.where(io == 2, w1n, jnp.where(io == 3, w2n, 0.0))))


def _route_meta_body(route_ref, pp_ref, be_ref):
    r = route_ref[...]
    e1 = r[:, 0:1].astype(jnp.int32)
    e2 = r[:, 1:2].astype(jnp.int32)
    lane = jax.lax.broadcasted_iota(jnp.int32, (SEQ, N_EXP), 1)
    oh1 = (lane == e1).astype(jnp.int32)
    oh2 = (lane == e2).astype(jnp.int32)
    tot = oh1 + oh2
    C = tot
    k = 1
    while k < SEQ:
        C = C + jnp.concatenate(
            [jnp.zeros((k, N_EXP), jnp.int32), C[:SEQ - k, :]], axis=0)
        k *= 2
    Cx = C - tot
    counts = C[SEQ - 1:SEQ, :]
    padded = ((counts + BLK - 1) // BLK) * BLK
    b = padded
    k = 1
    while k < N_EXP:
        b = b + jnp.concatenate(
            [jnp.zeros((1, k), jnp.int32), b[:, :N_EXP - k]], axis=1)
        k *= 2
    base = b - padded
    pos0 = (jnp.sum(Cx * oh1, axis=1, keepdims=True)
            + jnp.sum(base * oh1, axis=1, keepdims=True))
    pos1 = (jnp.sum((Cx + oh1) * oh2, axis=1, keepdims=True)
            + jnp.sum(base * oh2, axis=1, keepdims=True))
    pp_ref[...] = jnp.where(lane == 0, pos0, jnp.where(lane == 1, pos1, 0))
    jj = jax.lax.broadcasted_iota(jnp.int32, (64, N_EXP), 0) * BLK
    be = jnp.sum((jj >= base).astype(jnp.int32), axis=1, keepdims=True) - 1
    nbu = jnp.sum(padded, axis=1, keepdims=True) // BLK
    be = jnp.where(jax.lax.broadcasted_iota(jnp.int32, (64, 1), 0) < nbu,
                   be, N_EXP)
    be_ref[...] = be + jnp.zeros((64, N_EXP), jnp.int32)


@functools.cache
def _make_sc_dispatch():
    mesh = plsc.VectorSubcoreMesh(core_axis_name="c", subcore_axis_name="s")

    @functools.partial(
        pl.kernel, mesh=mesh,
        out_type=jax.ShapeDtypeStruct((P_MAX, D_MODEL), jnp.float32),
        scratch_types=[
            pltpu.VMEM((TPT,), jnp.int32),
            pltpu.VMEM((TPT,), jnp.int32),
            pltpu.VMEM((TPT, D_MODEL), jnp.float32),
            pltpu.SemaphoreType.DMA,
            pltpu.SemaphoreType.DMA,
        ],
    )
    def _sc_dispatch(hn_hbm, p0_hbm, p1_hbm, disp_hbm, p0_v, p1_v, rows_v,
                     sem1, sem2):
        wid = lax.axis_index("s") * 2 + lax.axis_index("c")
        base = wid * TPT
        pltpu.sync_copy(hn_hbm.at[pl.ds(base, TPT)], rows_v)
        pltpu.sync_copy(p0_hbm.at[pl.ds(base, TPT)], p0_v)
        pltpu.sync_copy(p1_hbm.at[pl.ds(base, TPT)], p1_v)
        d1 = pltpu.async_copy(rows_v, disp_hbm.at[p0_v], sem1)
        d2 = pltpu.async_copy(rows_v, disp_hbm.at[p1_v], sem2)
        d1.wait()
        d2.wait()

    return _sc_dispatch


@functools.cache
def _make_sc_combine():
    mesh = plsc.VectorSubcoreMesh(core_axis_name="c", subcore_axis_name="s")

    @functools.partial(
        pl.kernel, mesh=mesh,
        out_type=[jax.ShapeDtypeStruct((SEQ, D_MODEL), jnp.float32),
                  jax.ShapeDtypeStruct((SEQ, D_MODEL), jnp.float32)],
        scratch_types=[
            pltpu.VMEM((TPT,), jnp.int32),
            pltpu.VMEM((TPT, D_MODEL), jnp.float32),
            pltpu.SemaphoreType.DMA,
        ],
    )
    def _sc_combine(y_hbm, p0_hbm, p1_hbm, y0_hbm, y1_hbm, idx_v, rows_v, sem):
        wid = lax.axis_index("s") * 2 + lax.axis_index("c")
        base = wid * TPT
        pltpu.sync_copy(p0_hbm.at[pl.ds(base, TPT)], idx_v)
        pltpu.async_copy(y_hbm.at[idx_v], rows_v, sem).wait()
        pltpu.sync_copy(rows_v, y0_hbm.at[pl.ds(base, TPT)])
        pltpu.sync_copy(p1_hbm.at[pl.ds(base, TPT)], idx_v)
        pltpu.async_copy(y_hbm.at[idx_v], rows_v, sem).wait()
        pltpu.sync_copy(rows_v, y1_hbm.at[pl.ds(base, TPT)])

    return _sc_combine


def _ffn_body(be_ref, disp_ref, w1_ref, w3_ref, w2_ref, y_ref):
    b = pl.program_id(0)
    be = be_ref[b]

    @pl.when(be < N_EXP)
    def _go():
        xb = disp_ref[...].astype(jnp.bfloat16)
        dn = (((1,), (1,)), ((), ()))
        g1 = jax.lax.dot_general(xb, w1_ref[0], dn,
                                 preferred_element_type=jnp.float32)
        g3 = jax.lax.dot_general(xb, w3_ref[0], dn,
                                 preferred_element_type=jnp.float32)
        hcur = (_gelu(g1) * g3).astype(jnp.bfloat16)
        y_ref[...] = jax.lax.dot_general(hcur, w2_ref[0], dn,
                                         preferred_element_type=jnp.float32)


def _final_body(hn_ref, h_ref, route_ref, y0_ref, y1_ref,
                sw1_ref, sw2_ref, sw3_ref, out_ref):
    hn = hn_ref[...].astype(jnp.bfloat16)
    dn = (((1,), (1,)), ((), ()))
    s1 = jax.lax.dot_general(hn, sw1_ref[...], dn, preferred_element_type=jnp.float32)
    s3 = jax.lax.dot_general(hn, sw3_ref[...], dn, preferred_element_type=jnp.float32)
    sh = (_gelu(s1) * s3).astype(jnp.bfloat16)
    shared = jax.lax.dot_general(sh, sw2_ref[...], dn,
                                 preferred_element_type=jnp.float32)
    route = route_ref[...]
    out_ref[...] = (h_ref[...] + shared
                    + route[:, 2:3] * y0_ref[...]
                    + route[:, 3:4] * y1_ref[...])


def _rope_tabs():
    theta = 1.0 / (10000.0 ** (np.arange(0, HEAD_DIM, 2).astype(np.float32)
                               / HEAD_DIM))
    pos = np.arange(SEQ).astype(np.float32)
    fr = np.outer(pos, theta)
    cos, sin = np.cos(fr), np.sin(fr)
    ct = np.zeros((SEQ, D_MODEL), np.float32)
    st = np.zeros((SEQ, D_MODEL), np.float32)
    for h in range(N_HEADS):
        ct[:, h * 64 + 0:h * 64 + 64:2] = cos
        ct[:, h * 64 + 1:h * 64 + 64:2] = cos
        st[:, h * 64 + 0:h * 64 + 64:2] = -sin
        st[:, h * 64 + 1:h * 64 + 64:2] = sin
    return jnp.asarray(ct), jnp.asarray(st)


_CT, _ST = None, None


def kernel(x, q_W, kv_W, o_W, gate_W, w1, w2, w3, sw1, sw2, sw3):
    global _CT, _ST
    if _CT is None:
        _CT, _ST = _rope_tabs()
    B, S, D = x.shape
    x2 = x.reshape(S, D)
    perm = np.arange(D_MODEL).reshape(-1, 2)[:, ::-1].reshape(-1)
    q_Ws = q_W[perm]

    ao3 = pl.pallas_call(
        _attn_body,
        grid=(NBLK,),
        in_specs=[
            pl.BlockSpec((SBLK, D), lambda b: (b, 0)),
            pl.BlockSpec((D, D), lambda b: (0, 0)),
            pl.BlockSpec((D, D), lambda b: (0, 0)),
            pl.BlockSpec((2 * D, D), lambda b: (0, 0)),
            pl.BlockSpec((SBLK, D), lambda b: (b, 0)),
            pl.BlockSpec((SBLK, D), lambda b: (b, 0)),
        ],
        out_specs=pl.BlockSpec((N_HEADS, SBLK, HEAD_DIM), lambda b: (0, b, 0)),
        out_shape=jax.ShapeDtypeStruct((N_HEADS, SEQ, HEAD_DIM), jnp.float32),
    )(x2, q_W.astype(jnp.bfloat16), q_Ws.astype(jnp.bfloat16),
      kv_W.astype(jnp.bfloat16), _CT, _ST)
    ao_scr = ao3.reshape(S, D)

    h, hn, route = pl.pallas_call(
        _oroute_body,
        grid=(NBLK,),
        in_specs=[
            pl.BlockSpec((SBLK, D), lambda b: (b, 0)),
            pl.BlockSpec((SBLK, D), lambda b: (b, 0)),
            pl.BlockSpec((D, D), lambda b: (0, 0)),
            pl.BlockSpec((N_EXP, D), lambda b: (0, 0)),
        ],
        out_specs=[
            pl.BlockSpec((SBLK, D), lambda b: (b, 0)),
            pl.BlockSpec((SBLK, D), lambda b: (b, 0)),
            pl.BlockSpec((SBLK, N_EXP), lambda b: (b, 0)),
        ],
        out_shape=[
            jax.ShapeDtypeStruct((SEQ, D), jnp.float32),
            jax.ShapeDtypeStruct((SEQ, D), jnp.float32),
            jax.ShapeDtypeStruct((SEQ, N_EXP), jnp.float32),
        ],
    )(ao_scr, x2, o_W.astype(jnp.bfloat16), gate_W)

    pp, beo = pl.pallas_call(
        _route_meta_body,
        in_specs=[pl.BlockSpec((SEQ, N_EXP), lambda: (0, 0))],
        out_specs=[pl.BlockSpec((SEQ, N_EXP), lambda: (0, 0)),
                   pl.BlockSpec((64, N_EXP), lambda: (0, 0))],
        out_shape=[jax.ShapeDtypeStruct((SEQ, N_EXP), jnp.int32),
                   jax.ShapeDtypeStruct((64, N_EXP), jnp.int32)],
    )(route)
    p0 = pp[:, 0]
    p1 = pp[:, 1]
    block_expert = beo[:NB_FFN, 0]

    disp = _make_sc_dispatch()(hn, p0, p1)

    grid_spec = pltpu.PrefetchScalarGridSpec(
        num_scalar_prefetch=1,
        grid=(NB_FFN,),
        in_specs=[
            pl.BlockSpec((BLK, D), lambda b, be: (b, 0)),
            pl.BlockSpec((1, D_FF, D), lambda b, be: (jnp.minimum(be[b], N_EXP - 1), 0, 0)),
            pl.BlockSpec((1, D_FF, D), lambda b, be: (jnp.minimum(be[b], N_EXP - 1), 0, 0)),
            pl.BlockSpec((1, D, D_FF), lambda b, be: (jnp.minimum(be[b], N_EXP - 1), 0, 0)),
        ],
        out_specs=pl.BlockSpec((BLK, D), lambda b, be: (b, 0)),
    )
    y = pl.pallas_call(
        _ffn_body,
        grid_spec=grid_spec,
        out_shape=jax.ShapeDtypeStruct((P_MAX, D), jnp.float32),
    )(block_expert, disp, w1, w3, w2)

    y0s, y1s = _make_sc_combine()(y, p0, p1)

    out = pl.pallas_call(
        _final_body,
        grid=(NBLK,),
        in_specs=[
            pl.BlockSpec((SBLK, D), lambda b: (b, 0)),
            pl.BlockSpec((SBLK, D), lambda b: (b, 0)),
            pl.BlockSpec((SBLK, N_EXP), lambda b: (b, 0)),
            pl.BlockSpec((SBLK, D), lambda b: (b, 0)),
            pl.BlockSpec((SBLK, D), lambda b: (b, 0)),
            pl.BlockSpec((D_FF, D), lambda b: (0, 0)),
            pl.BlockSpec((D, D_FF), lambda b: (0, 0)),
            pl.BlockSpec((D_FF, D), lambda b: (0, 0)),
        ],
        out_specs=pl.BlockSpec((SBLK, D), lambda b: (b, 0)),
        out_shape=jax.ShapeDtypeStruct((SEQ, D), jnp.float32),
    )(hn, h, route, y0s, y1s,
      sw1.astype(jnp.bfloat16), sw2.astype(jnp.bfloat16),
      sw3.astype(jnp.bfloat16))

    return out.reshape(B, S, D)
